# baseline (speedup 1.0000x reference)
import functools

import jax
import jax.numpy as jnp
from jax import lax
from jax.experimental import pallas as pl
from jax.experimental.pallas import tpu as pltpu
from jax.experimental.pallas import tpu_sc as plsc


def _make_sc(V, D, B):
    info = plsc.get_sparse_core_info()
    NC, NS = info.num_cores, info.num_subcores
    NW = NC * NS
    assert B % NW == 0 and (B // NW) % 8 == 0
    b_per_w = B // NW
    K = 32
    NSEM = 8
    assert b_per_w % K == 0
    mesh = plsc.VectorSubcoreMesh(core_axis_name="c", subcore_axis_name="s")

    @functools.partial(
        pl.kernel,
        mesh=mesh,
        out_type=jax.ShapeDtypeStruct((B, D), jnp.float32),
        scratch_types=[
            pltpu.VMEM((b_per_w,), jnp.int32),
            pltpu.VMEM((b_per_w, D), jnp.float32),
        ] + [pltpu.SemaphoreType.DMA] * NSEM,
    )
    def k(table_hbm, idx_hbm, out_hbm, idx_v, rows_v, *sems):
        wid = lax.axis_index("s") * NC + lax.axis_index("c")
        base = wid * b_per_w
        pltpu.sync_copy(idx_hbm.at[pl.ds(base, b_per_w)], idx_v)

        def fire(g):
            handles = []
            for v in range(K // 16):
                ivec = idx_v[pl.ds(g * K + v * 16, 16)]
                for r in range(16):
                    dst = g * K + v * 16 + r
                    handles.append(
                        pltpu.async_copy(
                            table_hbm.at[pl.ds(ivec[r], 1), :],
                            rows_v.at[pl.ds(dst, 1), :],
                            sems[dst % NSEM],
                        )
                    )
            return handles

        n_chunks = b_per_w // K
        prev = fire(0)
        for g in range(1, n_chunks):
            cur = fire(g)
            for h in prev:
                h.wait()
            prev = cur
        for h in prev:
            h.wait()
        pltpu.sync_copy(rows_v, out_hbm.at[pl.ds(base, b_per_w)])

    return k


def _make_tc(V, D, B2):
    K = 32
    assert B2 % K == 0
    NCH = B2 // K

    def body(idx_smem, table_any, out_any, buf, sem, sem_out):
        def chunk(g, carry):
            for r in range(K):
                i = idx_smem[g * K + r]
                pltpu.make_async_copy(
                    table_any.at[pl.ds(i, 1), :],
                    buf.at[pl.ds(g * K + r, 1), :],
                    sem,
                ).start()

            @pl.when(g > 0)
            def _():
                for r in range(K):
                    pltpu.make_async_copy(
                        table_any.at[pl.ds(0, 1), :],
                        buf.at[pl.ds((g - 1) * K + r, 1), :],
                        sem,
                    ).wait()

            return carry

        lax.fori_loop(0, NCH, chunk, 0)
        for r in range(K):
            pltpu.make_async_copy(
                table_any.at[pl.ds(0, 1), :],
                buf.at[pl.ds((NCH - 1) * K + r, 1), :],
                sem,
            ).wait()
        out_copy = pltpu.make_async_copy(buf, out_any, sem_out)
        out_copy.start()
        out_copy.wait()

    grid_spec = pltpu.PrefetchScalarGridSpec(
        num_scalar_prefetch=1,
        grid=(1,),
        in_specs=[pl.BlockSpec(memory_space=pltpu.MemorySpace.HBM)],
        out_specs=pl.BlockSpec(memory_space=pltpu.MemorySpace.HBM),
        scratch_shapes=[
            pltpu.VMEM((B2, D), jnp.float32),
            pltpu.SemaphoreType.DMA,
            pltpu.SemaphoreType.DMA,
        ],
    )
    return pl.pallas_call(
        body,
        grid_spec=grid_spec,
        out_shape=jax.ShapeDtypeStruct((B2, D), jnp.float32),
    )


def kernel(full_tensor, indices):
    V, D = full_tensor.shape
    (B,) = indices.shape
    idx32 = indices.astype(jnp.int32)
    B_SC = 11264
    B_TC = B - B_SC
    sc_out = _make_sc(V, D, B_SC)(full_tensor, idx32[:B_SC])
    tc_out = _make_tc(V, D, B_TC)(idx32[B_SC:], full_tensor)
    return jnp.concatenate([sc_out, tc_out], axis=0)
